# baseline (speedup 1.0000x reference)
import functools

import jax
import jax.numpy as jnp
import numpy as np
from jax import lax
from jax.experimental import pallas as pl
from jax.experimental.pallas import tpu as pltpu
from jax.experimental.pallas import tpu_sc as plsc

_SPATIAL = [(64, 64), (32, 32), (16, 16), (8, 8)]
_NH, _NL, _NPT = 8, 4, 4
_M = _NL * 2 * _NPT
_LV = sum(h * w for h, w in _SPATIAL)
_STRIDE = 17
_BQ = 320
_GC = 2
_NCH = _LV // 16 // _GC

_OFF_SRC = np.zeros((1024,), np.int64)
_OFF_VALID = np.zeros((1024,), bool)
for _h in range(_NH):
    for _e in range(2):
        for _l in range(_NL):
            for _s in range(2):
                for _p in range(_NPT):
                    _c = _h * 128 + _e * 32 + _l * 8 + _s * 4 + _p
                    _OFF_SRC[_c] = _s * 256 + _h * 32 + _l * 8 + _p * 2 + _e
                    _OFF_VALID[_c] = True

_ATTN_SRC = np.zeros((1024,), np.int64)
_ATTN_VALID = np.zeros((1024,), bool)
for _h in range(_NH):
    for _m in range(_M):
        _ATTN_SRC[_h * 128 + _m] = _h * 32 + _m
        _ATTN_VALID[_h * 128 + _m] = True

_S_SUM = np.zeros((1024, 8), np.float32)
_S_BCAST = np.zeros((8, 1024), np.float32)
for _c in range(1024):
    if _ATTN_VALID[_c]:
        _S_SUM[_c, _c // 128] = 1.0
for _h in range(_NH):
    _S_BCAST[_h, _h * 128:(_h + 1) * 128] = 1.0

_starts = np.cumsum([0] + [h * w for h, w in _SPATIAL])[:4]


def _proj_body(qc_ref, val_ref, woff_ref, boff_ref, wattn_ref,
               battn_ref, wval_ref, bval_ref, ssum_ref, sbc_ref,
               off_ref, aw_ref, vproj_ref):
    qc = qc_ref[0]
    off_ref[0] = jnp.dot(qc, woff_ref[...],
                         preferred_element_type=jnp.float32) + boff_ref[0]
    attn = jnp.dot(qc, wattn_ref[...], preferred_element_type=jnp.float32)
    attn = attn + battn_ref[0]
    e = jnp.exp(attn)
    denom = jnp.dot(e, ssum_ref[...], preferred_element_type=jnp.float32,
                    precision=lax.Precision.HIGHEST)
    aw_ref[0] = e * jnp.dot(1.0 / denom, sbc_ref[...],
                            preferred_element_type=jnp.float32,
                            precision=lax.Precision.HIGHEST)
    vproj_ref[0] = (jnp.dot(val_ref[0], wval_ref[...],
                            preferred_element_type=jnp.float32) + bval_ref[0])


def _run_proj(qc, value, woffp, boffp, wattnp, battnp, wval, bval):
    bs, Lq, _ = value.shape
    grid = (bs, Lq // _BQ)
    row_blk = lambda d: pl.BlockSpec((1, _BQ, d), lambda b, i: (b, i, 0))
    full = lambda s: pl.BlockSpec(s, lambda b, i: tuple(0 for _ in s))
    f32 = jnp.float32
    outs = [
        jax.ShapeDtypeStruct((bs, Lq, 1024), f32),
        jax.ShapeDtypeStruct((bs, Lq, 1024), f32),
        jax.ShapeDtypeStruct((bs, Lq, 256), f32),
    ]
    return pl.pallas_call(
        _proj_body,
        grid=grid,
        in_specs=[row_blk(512), row_blk(256),
                  full((512, 1024)), full((1, 1024)),
                  full((512, 1024)), full((1, 1024)),
                  full((256, 256)), full((1, 256)),
                  full((1024, 8)), full((8, 1024))],
        out_specs=[row_blk(1024), row_blk(1024), row_blk(256)],
        out_shape=outs,
    )(qc, value, woffp, boffp, wattnp, battnp, wval, bval,
      jnp.asarray(_S_SUM), jnp.asarray(_S_BCAST))


def _floor(x):
    xf = lax.convert_element_type(lax.convert_element_type(x, jnp.int32),
                                  jnp.float32)
    return jnp.where(xf > x, xf - 1.0, xf)


def _sc_body(v_hbm, off_hbm, aw_hbm, rp_hbm, out_hbm,
             table_v, off_v, aw_v, rp_v, out_v,
             sr0, so0, sa0, sr1, so1, sa1):
    cid = lax.axis_index("c")
    sid = lax.axis_index("s")
    w = sid * 2 + cid
    b = w // _NH
    h = w % _NH
    qi = lax.iota(jnp.int32, 16)
    GQ = _GC * 16
    sems = ((sr0, so0, sa0), (sr1, so1, sa1))

    def copies(c, par):
        q0 = c * GQ
        pslice = pl.ds(par * GQ, GQ)
        return (
            pltpu.make_async_copy(rp_hbm.at[b, pl.ds(q0, GQ)],
                                  rp_v.at[pslice], sems[par][0]),
            pltpu.make_async_copy(
                off_hbm.at[b, pl.ds(q0, GQ), pl.ds(h * 128, 128)],
                off_v.at[pslice], sems[par][1]),
            pltpu.make_async_copy(
                aw_hbm.at[b, pl.ds(q0, GQ), pl.ds(h * 128, 128)],
                aw_v.at[pslice], sems[par][2]),
        )

    def half(t, _):
        pltpu.sync_copy(v_hbm.at[w, t], table_v)
        for d in copies(0, 0):
            d.start()

        def compute(c, par):
            for r in range(_GC * 16):
                out_v[r] = jnp.zeros((16,), jnp.float32)
            for g in range(_GC):
                rows = qi + (par * _GC + g) * 16
                orows = qi + g * 16
                for l, (H, W) in enumerate(_SPATIAL):
                    rpx = plsc.load_gather(
                        rp_v, [rows, jnp.full((16,), 2 * l, jnp.int32)])
                    rpy = plsc.load_gather(
                        rp_v, [rows, jnp.full((16,), 2 * l + 1, jnp.int32)])
                    xb = rpx - 0.5
                    yb = rpy - 0.5

                    def sample(sp, carry):
                        cx = jnp.full((16,), l * 8, jnp.int32) + sp
                        x = xb + plsc.load_gather(off_v, [rows, cx])
                        y = yb + plsc.load_gather(off_v, [rows, cx + 32])
                        a = plsc.load_gather(aw_v, [rows, cx])
                        x0 = _floor(x)
                        fx = x - x0
                        y0 = _floor(y)
                        fy = y - y0
                        bx = jnp.clip(x0, 0.0, float(W - 2))
                        in01 = (x0 >= 0.0) & (x0 <= float(W - 2))
                        wb0 = jnp.where(in01, 1.0 - fx, 0.0) \
                            + jnp.where(x0 == -1.0, fx, 0.0)
                        wb1 = jnp.where(in01, fx, 0.0) \
                            + jnp.where(x0 == float(W - 1), 1.0 - fx, 0.0)
                        for k in (0, 1):
                            yk = y0 + float(k)
                            vy = (yk >= 0.0) & (yk <= float(H - 1))
                            yc = jnp.clip(yk, 0.0, float(H - 1))
                            wyk = jnp.where(vy, 1.0 - jnp.abs(fy - float(k)),
                                            0.0) * a
                            w0 = wyk * wb0
                            w1 = wyk * wb1
                            base = (yc * float(_STRIDE * W)
                                    + bx * float(_STRIDE)
                                    + float(_STRIDE * _starts[l])
                                    ).astype(jnp.int32)
                            for j in range(16):
                                a0 = base + j
                                g0v = plsc.load_gather(table_v, [a0])
                                g1v = plsc.load_gather(table_v, [a0 + _STRIDE])
                                plsc.addupdate_scatter(
                                    out_v,
                                    [orows, jnp.full((16,), j, jnp.int32)],
                                    w0 * g0v + w1 * g1v)
                        return carry

                    lax.fori_loop(0, 8, sample, 0)
            pltpu.sync_copy(
                out_v, out_hbm.at[b, h * 2 + t, pl.ds(c * GQ, GQ)])

        def two_chunks(i, _):
            c0 = i * 2
            for d in copies(c0 + 1, 1):
                d.start()
            for d in copies(c0, 0):
                d.wait()
            compute(c0, 0)

            @pl.when(c0 + 2 < _NCH)
            def _():
                for d in copies(c0 + 2, 0):
                    d.start()

            for d in copies(c0 + 1, 1):
                d.wait()
            compute(c0 + 1, 1)
            return 0

        lax.fori_loop(0, _NCH // 2, two_chunks, 0)
        return 0

    lax.fori_loop(0, 2, half, 0)


def _run_sc(v_sc, off, aw, rp_p):
    bs, Lq, _ = off.shape
    mesh = plsc.VectorSubcoreMesh(core_axis_name="c", subcore_axis_name="s")
    kern = pl.kernel(
        _sc_body,
        out_type=jax.ShapeDtypeStruct((bs, 2 * _NH, Lq, 16), jnp.float32),
        mesh=mesh,
        compiler_params=pltpu.CompilerParams(needs_layout_passes=False),
        scratch_types=[
            pltpu.VMEM((_LV * _STRIDE,), jnp.float32),
            pltpu.VMEM((2 * _GC * 16, 128), jnp.float32),
            pltpu.VMEM((2 * _GC * 16, 128), jnp.float32),
            pltpu.VMEM((2 * _GC * 16, 16), jnp.float32),
            pltpu.VMEM((_GC * 16, 16), jnp.float32),
            pltpu.SemaphoreType.DMA,
            pltpu.SemaphoreType.DMA,
            pltpu.SemaphoreType.DMA,
            pltpu.SemaphoreType.DMA,
            pltpu.SemaphoreType.DMA,
            pltpu.SemaphoreType.DMA,
        ],
    )
    return kern(v_sc, off, aw, rp_p)


def _final_body(x_ref, w_ref, b_ref, o_ref):
    x = x_ref[0]
    o = b_ref[0]
    for c2 in range(16):
        hh, tt = c2 // 2, c2 % 2
        r0 = hh * 32 + tt * 16
        o = o + jnp.dot(x[c2], w_ref[r0:r0 + 16, :],
                        preferred_element_type=jnp.float32)
    o_ref[0] = o


def _run_final(x, wout, bout):
    bs, _, Lq, _ = x.shape
    Cd = 256
    return pl.pallas_call(
        _final_body,
        grid=(bs, Lq // _BQ),
        in_specs=[pl.BlockSpec((1, 16, _BQ, 16), lambda b, i: (b, 0, i, 0)),
                  pl.BlockSpec((Cd, Cd), lambda b, i: (0, 0)),
                  pl.BlockSpec((1, Cd), lambda b, i: (0, 0))],
        out_specs=pl.BlockSpec((1, _BQ, Cd), lambda b, i: (b, i, 0)),
        out_shape=jax.ShapeDtypeStruct((bs, Lq, Cd), jnp.float32),
    )(x, wout, bout.reshape(1, Cd))


def kernel(query, query_pos, reference_points, value, value_spatial_shapes,
           W_off, b_off, W_attn, b_attn, W_val, b_val, W_out, b_out):
    bs, Lq, Cd = query.shape
    qc = jnp.concatenate([query, query_pos], axis=-1)
    scale = np.zeros((2 * _NL,), np.float32)
    for l, (H, W) in enumerate(_SPATIAL):
        scale[2 * l] = W
        scale[2 * l + 1] = H
    rp = reference_points.reshape(bs, Lq, 2 * _NL) * jnp.asarray(scale)
    rp_p = jnp.pad(rp, ((0, 0), (0, 0), (0, 8)))

    woffp = jnp.where(jnp.asarray(_OFF_VALID), W_off[:, _OFF_SRC], 0.0)
    boffp = jnp.where(jnp.asarray(_OFF_VALID), b_off[_OFF_SRC],
                      0.0).reshape(1, 1024)
    wattnp = jnp.where(jnp.asarray(_ATTN_VALID), W_attn[:, _ATTN_SRC], 0.0)
    battnp = jnp.where(jnp.asarray(_ATTN_VALID), b_attn[_ATTN_SRC],
                       0.0).reshape(1, 1024)

    off, aw, vproj = _run_proj(qc, value, woffp, boffp, wattnp, battnp,
                               W_val, b_val.reshape(1, 256))

    vp = vproj.reshape(bs, Lq, _NH, 2, 16).transpose(0, 2, 3, 1, 4)
    vp = jnp.pad(vp, ((0, 0), (0, 0), (0, 0), (0, 0), (0, 1)))
    v_sc = vp.reshape(bs * _NH, 2, Lq * _STRIDE)

    out = _run_sc(v_sc, off, aw, rp_p)
    return _run_final(out, W_out, b_out)


# speedup vs baseline: 3.8229x; 3.8229x over previous
import functools

import jax
import jax.numpy as jnp
import numpy as np
from jax import lax
from jax.experimental import pallas as pl
from jax.experimental.pallas import tpu as pltpu
from jax.experimental.pallas import tpu_sc as plsc

_SPATIAL = [(64, 64), (32, 32), (16, 16), (8, 8)]
_NH, _NL, _NPT = 8, 4, 4
_M = _NL * 2 * _NPT
_LV = sum(h * w for h, w in _SPATIAL)
_STRIDE = 17
_BQ = 320
_GC = 2
_NCH = _LV // 16 // _GC

_OFF_SRC = np.zeros((1024,), np.int64)
_OFF_VALID = np.zeros((1024,), bool)
for _h in range(_NH):
    for _e in range(2):
        for _l in range(_NL):
            for _s in range(2):
                for _p in range(_NPT):
                    _c = _h * 128 + _e * 32 + _l * 8 + _s * 4 + _p
                    _OFF_SRC[_c] = _s * 256 + _h * 32 + _l * 8 + _p * 2 + _e
                    _OFF_VALID[_c] = True

_ATTN_SRC = np.zeros((1024,), np.int64)
_ATTN_VALID = np.zeros((1024,), bool)
for _h in range(_NH):
    for _m in range(_M):
        _ATTN_SRC[_h * 128 + _m] = _h * 32 + _m
        _ATTN_VALID[_h * 128 + _m] = True

_S_SUM = np.zeros((1024, 8), np.float32)
_S_BCAST = np.zeros((8, 1024), np.float32)
for _c in range(1024):
    if _ATTN_VALID[_c]:
        _S_SUM[_c, _c // 128] = 1.0
for _h in range(_NH):
    _S_BCAST[_h, _h * 128:(_h + 1) * 128] = 1.0

_starts = np.cumsum([0] + [h * w for h, w in _SPATIAL])[:4]


def _proj_body(qc_ref, val_ref, woff_ref, boff_ref, wattn_ref,
               battn_ref, wval_ref, bval_ref, ssum_ref, sbc_ref,
               off_ref, aw_ref, vproj_ref):
    qc = qc_ref[0]
    off_ref[0] = jnp.dot(qc, woff_ref[...],
                         preferred_element_type=jnp.float32) + boff_ref[0]
    attn = jnp.dot(qc, wattn_ref[...], preferred_element_type=jnp.float32)
    attn = attn + battn_ref[0]
    e = jnp.exp(attn)
    denom = jnp.dot(e, ssum_ref[...], preferred_element_type=jnp.float32,
                    precision=lax.Precision.HIGHEST)
    aw_ref[0] = e * jnp.dot(1.0 / denom, sbc_ref[...],
                            preferred_element_type=jnp.float32,
                            precision=lax.Precision.HIGHEST)
    vproj_ref[0] = (jnp.dot(val_ref[0], wval_ref[...],
                            preferred_element_type=jnp.float32) + bval_ref[0])


def _run_proj(qc, value, woffp, boffp, wattnp, battnp, wval, bval):
    bs, Lq, _ = value.shape
    grid = (bs, Lq // _BQ)
    row_blk = lambda d: pl.BlockSpec((1, _BQ, d), lambda b, i: (b, i, 0))
    full = lambda s: pl.BlockSpec(s, lambda b, i: tuple(0 for _ in s))
    f32 = jnp.float32
    outs = [
        jax.ShapeDtypeStruct((bs, Lq, 1024), f32),
        jax.ShapeDtypeStruct((bs, Lq, 1024), f32),
        jax.ShapeDtypeStruct((bs, Lq, 256), f32),
    ]
    return pl.pallas_call(
        _proj_body,
        grid=grid,
        in_specs=[row_blk(512), row_blk(256),
                  full((512, 1024)), full((1, 1024)),
                  full((512, 1024)), full((1, 1024)),
                  full((256, 256)), full((1, 256)),
                  full((1024, 8)), full((8, 1024))],
        out_specs=[row_blk(1024), row_blk(1024), row_blk(256)],
        out_shape=outs,
    )(qc, value, woffp, boffp, wattnp, battnp, wval, bval,
      jnp.asarray(_S_SUM), jnp.asarray(_S_BCAST))


def _floor(x):
    xf = lax.convert_element_type(lax.convert_element_type(x, jnp.int32),
                                  jnp.float32)
    return jnp.where(xf > x, xf - 1.0, xf)


def _sc_body(v_hbm, off_hbm, aw_hbm, rp_hbm, out_hbm,
             table_v, off_v, aw_v, rp_v, out_v,
             sr0, so0, sa0, sr1, so1, sa1):
    cid = lax.axis_index("c")
    sid = lax.axis_index("s")
    w = sid * 2 + cid
    b = w // _NH
    h = w % _NH
    qi = lax.iota(jnp.int32, 16)
    GQ = _GC * 16
    sems = ((sr0, so0, sa0), (sr1, so1, sa1))

    def copies(c, par):
        q0 = c * GQ
        pslice = pl.ds(par * GQ, GQ)
        return (
            pltpu.make_async_copy(rp_hbm.at[b, pl.ds(q0, GQ)],
                                  rp_v.at[pslice], sems[par][0]),
            pltpu.make_async_copy(
                off_hbm.at[b, pl.ds(q0, GQ), pl.ds(h * 128, 128)],
                off_v.at[pslice], sems[par][1]),
            pltpu.make_async_copy(
                aw_hbm.at[b, pl.ds(q0, GQ), pl.ds(h * 128, 128)],
                aw_v.at[pslice], sems[par][2]),
        )

    def half(t, _):
        pltpu.sync_copy(v_hbm.at[w, t], table_v)
        for d in copies(0, 0):
            d.start()

        def compute(c, par):
            for g in range(_GC):
                rows = qi + (par * _GC + g) * 16
                orows = qi + g * 16
                accs = tuple(jnp.zeros((16,), jnp.float32) for _ in range(16))
                for l, (H, W) in enumerate(_SPATIAL):
                    rpx = plsc.load_gather(
                        rp_v, [rows, jnp.full((16,), 2 * l, jnp.int32)])
                    rpy = plsc.load_gather(
                        rp_v, [rows, jnp.full((16,), 2 * l + 1, jnp.int32)])
                    xb = rpx - 0.5
                    yb = rpy - 0.5

                    def sample(sp, accs):
                        cx = jnp.full((16,), l * 8, jnp.int32) + sp
                        x = xb + plsc.load_gather(off_v, [rows, cx])
                        y = yb + plsc.load_gather(off_v, [rows, cx + 32])
                        a = plsc.load_gather(aw_v, [rows, cx])
                        x0 = _floor(x)
                        fx = x - x0
                        y0 = _floor(y)
                        fy = y - y0
                        bx = jnp.clip(x0, 0.0, float(W - 2))
                        in01 = (x0 >= 0.0) & (x0 <= float(W - 2))
                        wb0 = jnp.where(in01, 1.0 - fx, 0.0) \
                            + jnp.where(x0 == -1.0, fx, 0.0)
                        wb1 = jnp.where(in01, fx, 0.0) \
                            + jnp.where(x0 == float(W - 1), 1.0 - fx, 0.0)
                        new = list(accs)
                        for k in (0, 1):
                            yk = y0 + float(k)
                            vy = (yk >= 0.0) & (yk <= float(H - 1))
                            yc = jnp.clip(yk, 0.0, float(H - 1))
                            wyk = jnp.where(vy, 1.0 - jnp.abs(fy - float(k)),
                                            0.0) * a
                            w0 = wyk * wb0
                            w1 = wyk * wb1
                            base = (yc * float(_STRIDE * W)
                                    + bx * float(_STRIDE)
                                    + float(_STRIDE * _starts[l])
                                    ).astype(jnp.int32)
                            for j in range(16):
                                a0 = base + j
                                g0v = plsc.load_gather(table_v, [a0])
                                g1v = plsc.load_gather(table_v, [a0 + _STRIDE])
                                new[j] = new[j] + w0 * g0v + w1 * g1v
                        return tuple(new)

                    accs = lax.fori_loop(0, 8, sample, accs)
                for j in range(16):
                    plsc.store_scatter(
                        out_v, [orows, jnp.full((16,), j, jnp.int32)], accs[j])
            pltpu.sync_copy(
                out_v, out_hbm.at[b, h * 2 + t, pl.ds(c * GQ, GQ)])

        def two_chunks(i, _):
            c0 = i * 2
            for d in copies(c0 + 1, 1):
                d.start()
            for d in copies(c0, 0):
                d.wait()
            compute(c0, 0)

            @pl.when(c0 + 2 < _NCH)
            def _():
                for d in copies(c0 + 2, 0):
                    d.start()

            for d in copies(c0 + 1, 1):
                d.wait()
            compute(c0 + 1, 1)
            return 0

        lax.fori_loop(0, _NCH // 2, two_chunks, 0)
        return 0

    lax.fori_loop(0, 2, half, 0)


def _run_sc(v_sc, off, aw, rp_p):
    bs, Lq, _ = off.shape
    mesh = plsc.VectorSubcoreMesh(core_axis_name="c", subcore_axis_name="s")
    kern = pl.kernel(
        _sc_body,
        out_type=jax.ShapeDtypeStruct((bs, 2 * _NH, Lq, 16), jnp.float32),
        mesh=mesh,
        compiler_params=pltpu.CompilerParams(needs_layout_passes=False),
        scratch_types=[
            pltpu.VMEM((_LV * _STRIDE,), jnp.float32),
            pltpu.VMEM((2 * _GC * 16, 128), jnp.float32),
            pltpu.VMEM((2 * _GC * 16, 128), jnp.float32),
            pltpu.VMEM((2 * _GC * 16, 16), jnp.float32),
            pltpu.VMEM((_GC * 16, 16), jnp.float32),
            pltpu.SemaphoreType.DMA,
            pltpu.SemaphoreType.DMA,
            pltpu.SemaphoreType.DMA,
            pltpu.SemaphoreType.DMA,
            pltpu.SemaphoreType.DMA,
            pltpu.SemaphoreType.DMA,
        ],
    )
    return kern(v_sc, off, aw, rp_p)


def _final_body(x_ref, w_ref, b_ref, o_ref):
    x = x_ref[0]
    o = b_ref[0]
    for c2 in range(16):
        hh, tt = c2 // 2, c2 % 2
        r0 = hh * 32 + tt * 16
        o = o + jnp.dot(x[c2], w_ref[r0:r0 + 16, :],
                        preferred_element_type=jnp.float32)
    o_ref[0] = o


def _run_final(x, wout, bout):
    bs, _, Lq, _ = x.shape
    Cd = 256
    return pl.pallas_call(
        _final_body,
        grid=(bs, Lq // _BQ),
        in_specs=[pl.BlockSpec((1, 16, _BQ, 16), lambda b, i: (b, 0, i, 0)),
                  pl.BlockSpec((Cd, Cd), lambda b, i: (0, 0)),
                  pl.BlockSpec((1, Cd), lambda b, i: (0, 0))],
        out_specs=pl.BlockSpec((1, _BQ, Cd), lambda b, i: (b, i, 0)),
        out_shape=jax.ShapeDtypeStruct((bs, Lq, Cd), jnp.float32),
    )(x, wout, bout.reshape(1, Cd))


def kernel(query, query_pos, reference_points, value, value_spatial_shapes,
           W_off, b_off, W_attn, b_attn, W_val, b_val, W_out, b_out):
    bs, Lq, Cd = query.shape
    qc = jnp.concatenate([query, query_pos], axis=-1)
    scale = np.zeros((2 * _NL,), np.float32)
    for l, (H, W) in enumerate(_SPATIAL):
        scale[2 * l] = W
        scale[2 * l + 1] = H
    rp = reference_points.reshape(bs, Lq, 2 * _NL) * jnp.asarray(scale)
    rp_p = jnp.pad(rp, ((0, 0), (0, 0), (0, 8)))

    woffp = jnp.where(jnp.asarray(_OFF_VALID), W_off[:, _OFF_SRC], 0.0)
    boffp = jnp.where(jnp.asarray(_OFF_VALID), b_off[_OFF_SRC],
                      0.0).reshape(1, 1024)
    wattnp = jnp.where(jnp.asarray(_ATTN_VALID), W_attn[:, _ATTN_SRC], 0.0)
    battnp = jnp.where(jnp.asarray(_ATTN_VALID), b_attn[_ATTN_SRC],
                       0.0).reshape(1, 1024)

    off, aw, vproj = _run_proj(qc, value, woffp, boffp, wattnp, battnp,
                               W_val, b_val.reshape(1, 256))

    vp = vproj.reshape(bs, Lq, _NH, 2, 16).transpose(0, 2, 3, 1, 4)
    vp = jnp.pad(vp, ((0, 0), (0, 0), (0, 0), (0, 0), (0, 1)))
    v_sc = vp.reshape(bs * _NH, 2, Lq * _STRIDE)

    out = _run_sc(v_sc, off, aw, rp_p)
    return _run_final(out, W_out, b_out)
